# Initial kernel scaffold; baseline (speedup 1.0000x reference)
import jax
import jax.numpy as jnp
from jax.experimental import pallas as pl


def kernel(node_features, edge_index, edge_features, od_pairs, traffic_stats, W_node, b_node, W_edge, b_edge, Wm0, bm0, Wu0, bu0, Wm1, bm1, Wu1, bu1, Ws1, bs1, Ws2, bs2, Wk1, bk1, Wk2, bk2, Wk3, bk3):
    raise NotImplementedError("write your pallas kernel here")


# speedup vs baseline: 2.5991x; 2.5991x over previous
import functools

import jax
import jax.numpy as jnp
from jax import lax
from jax.experimental import pallas as pl
from jax.experimental.pallas import tpu as pltpu
from jax.experimental.pallas import tpu_sc as plsc

_NN = 10000
_NE = 320000
_DF = 128
_DE = 16
_H = 64
_NOD = 50000

_NC = 2
_NS = 16
_NW = _NC * _NS

_EROWS = _NE // 128
_ERPW = _EROWS // _NW
_EXTRA = _EROWS - _ERPW * _NW
_TPS = 624
_TREM = _NN - _TPS * _NS

_NODP = 50176
_OROWS = _NODP // 128
_ORPW = _OROWS // _NW
_OXTRA = _OROWS - _ORPW * _NW


@functools.lru_cache(maxsize=None)
def _sc_mesh():
    return plsc.VectorSubcoreMesh(core_axis_name="c", subcore_axis_name="s",
                                  num_cores=_NC, num_subcores=_NS)


def _node_embed_body(x_ref, wn_ref, bn_ref, wma_ref, h_ref, g_ref):
    h = jnp.maximum(jnp.dot(x_ref[...], wn_ref[...],
                            preferred_element_type=jnp.float32) + bn_ref[...], 0.0)
    h_ref[...] = h
    g = jnp.dot(h, wma_ref[...], preferred_element_type=jnp.float32)
    g_ref[...] = jnp.concatenate([g, jnp.zeros_like(g)], axis=1)


def _node_embed(x, wn, bn, wma):
    return pl.pallas_call(
        _node_embed_body,
        out_shape=(jax.ShapeDtypeStruct((_NN, _H), jnp.float32),
                   jax.ShapeDtypeStruct((_NN, 2 * _H), jnp.float32)),
    )(x, wn, bn, wma)


_EBLK = 4000


def _edge_embed_body(ef_ref, we_ref, be_ref, wb0_ref, bm0_ref, wb1_ref, bm1_ref,
                     c_ref):
    e = jnp.maximum(jnp.dot(ef_ref[...], we_ref[...],
                            preferred_element_type=jnp.float32) + be_ref[...], 0.0)
    c0 = jnp.dot(e, wb0_ref[...], preferred_element_type=jnp.float32) + bm0_ref[...]
    c1 = jnp.dot(e, wb1_ref[...], preferred_element_type=jnp.float32) + bm1_ref[...]
    c_ref[...] = jnp.concatenate([c0, c1], axis=1)


def _edge_embed(ef, we, be, wb0, bm0, wb1, bm1):
    nblk = _NE // _EBLK
    return pl.pallas_call(
        _edge_embed_body,
        grid=(nblk,),
        in_specs=[pl.BlockSpec((_EBLK, _DE), lambda i: (i, 0)),
                  pl.BlockSpec((_DE, _H), lambda i: (0, 0)),
                  pl.BlockSpec((1, _H), lambda i: (0, 0)),
                  pl.BlockSpec((_H, _H), lambda i: (0, 0)),
                  pl.BlockSpec((1, _H), lambda i: (0, 0)),
                  pl.BlockSpec((_H, _H), lambda i: (0, 0)),
                  pl.BlockSpec((1, _H), lambda i: (0, 0))],
        out_specs=pl.BlockSpec((_EBLK, 2 * _H), lambda i: (i, 0)),
        out_shape=jax.ShapeDtypeStruct((_NE, 2 * _H), jnp.float32),
    )(ef, we, be, wb0, bm0, wb1, bm1)


def _update0_body(h_ref, agg_ref, wua_ref, wub_ref, bu_ref, wma_ref, h1_ref, g1_ref):
    aggsum = agg_ref[0, :, :_H] + agg_ref[1, :, :_H]
    h1 = jnp.maximum(jnp.dot(h_ref[...], wua_ref[...], preferred_element_type=jnp.float32)
                     + jnp.dot(aggsum, wub_ref[...], preferred_element_type=jnp.float32)
                     + bu_ref[...], 0.0)
    h1_ref[...] = h1
    g1 = jnp.dot(h1, wma_ref[...], preferred_element_type=jnp.float32)
    g1_ref[...] = jnp.concatenate([jnp.zeros_like(g1), g1], axis=1)


def _update0(h, agg, wua, wub, bu, wma):
    return pl.pallas_call(
        _update0_body,
        out_shape=(jax.ShapeDtypeStruct((_NN, _H), jnp.float32),
                   jax.ShapeDtypeStruct((_NN, 2 * _H), jnp.float32)),
    )(h, agg, wua, wub, bu, wma)


def _update1_body(h_ref, agg_ref, wua_ref, wub_ref, bu_ref,
                  ws1a_ref, ws1b_ref, bs1_ref, ts_ref,
                  wk1a_ref, wk1b_ref, bk1_ref, wk2_ref, bk2_ref, wk3_ref, bk3_ref,
                  ab_ref, kc_ref, ki_ref):
    aggsum = agg_ref[0, :, _H:] + agg_ref[1, :, _H:]
    h2 = jnp.maximum(jnp.dot(h_ref[...], wua_ref[...], preferred_element_type=jnp.float32)
                     + jnp.dot(aggsum, wub_ref[...], preferred_element_type=jnp.float32)
                     + bu_ref[...], 0.0)
    a = jnp.dot(h2, ws1a_ref[...], preferred_element_type=jnp.float32) + bs1_ref[...]
    b = jnp.dot(h2, ws1b_ref[...], preferred_element_type=jnp.float32)
    ab_ref[...] = jnp.concatenate([a, b], axis=1)
    ge = jnp.sum(h2, axis=0, keepdims=True) * (1.0 / _NN)
    x1 = jnp.maximum(jnp.dot(ge, wk1a_ref[...], preferred_element_type=jnp.float32)
                     + jnp.dot(ts_ref[...], wk1b_ref[...], preferred_element_type=jnp.float32)
                     + bk1_ref[...], 0.0)
    x2 = jnp.maximum(jnp.dot(x1, wk2_ref[...], preferred_element_type=jnp.float32)
                     + bk2_ref[...], 0.0)
    raw = jnp.dot(x2, wk3_ref[...], preferred_element_type=jnp.float32) + bk3_ref[...]
    kc = 1.0 + jax.nn.sigmoid(raw) * 49.0
    kc_ref[...] = kc
    ki_ref[...] = jnp.clip(jnp.round(kc), 1.0, 50.0)


def _update1(h, agg, wua, wub, bu, ws1a, ws1b, bs1, ts,
             wk1a, wk1b, bk1, wk2, bk2, wk3, bk3):
    return pl.pallas_call(
        _update1_body,
        out_shape=(jax.ShapeDtypeStruct((_NN, 2 * _H), jnp.float32),
                   jax.ShapeDtypeStruct((1, 1), jnp.float32),
                   jax.ShapeDtypeStruct((1, 1), jnp.float32)),
    )(h, agg, wua, wub, bu, ws1a, ws1b, bs1, ts,
      wk1a, wk1b, bk1, wk2, bk2, wk3, bk3)


def _make_edge_body(qlo):
    def body(src_hbm, dst_hbm, c_hbm, g_hbm, out_hbm,
             agg_s, src_x, dst_x, c_v, rows_v, gsem):
        cid = lax.axis_index("c")
        sid = lax.axis_index("s")
        wid = cid * _NS + sid
        base = _ERPW * wid

        @pl.loop(0, 128)
        def _zero(i):
            for q in range(8):
                rows_v[i, pl.ds(q * 16, 16)] = jnp.zeros((16,), jnp.float32)

        for k, sz in ((0, 128), (128, 128), (256, 128), (384, 128), (512, 112)):
            pltpu.sync_copy(rows_v.at[pl.ds(0, sz)],
                            agg_s.at[pl.ds(sid * _TPS + k, sz)])

        @pl.when(sid == 0)
        def _zrem():
            pltpu.sync_copy(rows_v.at[pl.ds(0, _TREM)],
                            agg_s.at[pl.ds(_TPS * _NS, _TREM)])

        plsc.subcore_barrier()

        def _process(row):
            pltpu.sync_copy(src_hbm.at[pl.ds(row, 1)], src_x)
            pltpu.sync_copy(dst_hbm.at[pl.ds(row, 1)], dst_x)
            pltpu.sync_copy(c_hbm.at[pl.ds(row * 128, 128)], c_v)
            pltpu.async_copy(g_hbm.at[src_x.at[0, 0]], rows_v, gsem).wait()

            @pl.loop(0, 128)
            def _relu(j):
                for q in range(qlo, qlo + 4):
                    s = pl.ds(q * 16, 16)
                    rows_v[j, s] = jnp.maximum(rows_v[j, s] + c_v[j, s], 0.0)

            pltpu.sync_copy(rows_v, agg_s.at[dst_x.at[0, 0]], add=True)

        @pl.loop(0, _ERPW)
        def _main(r):
            _process(base + r)

        @pl.when(wid < _EXTRA)
        def _tail():
            _process(_ERPW * _NW + wid)

        plsc.subcore_barrier()
        pltpu.sync_copy(agg_s.at[pl.ds(sid * _TPS, _TPS)],
                        out_hbm.at[cid, pl.ds(sid * _TPS, _TPS)])

        @pl.when(sid == 0)
        def _orem():
            pltpu.sync_copy(agg_s.at[pl.ds(_TPS * _NS, _TREM)],
                            out_hbm.at[cid, pl.ds(_TPS * _NS, _TREM)])

    return body


@functools.lru_cache(maxsize=None)
def _edge_pass_kernel(qlo):
    return pl.kernel(
        _make_edge_body(qlo),
        out_type=jax.ShapeDtypeStruct((_NC, _NN, 2 * _H), jnp.float32),
        mesh=_sc_mesh(),
        scratch_types=[
            pltpu.VMEM_SHARED((_NN, 2 * _H), jnp.float32),
            pltpu.VMEM((1, 1, 128), jnp.int32),
            pltpu.VMEM((1, 1, 128), jnp.int32),
            pltpu.VMEM((128, 2 * _H), jnp.float32),
            pltpu.VMEM((128, 2 * _H), jnp.float32),
            pltpu.SemaphoreType.DMA,
        ],
    )


def _od_sc_body(idxa_hbm, idxb_hbm, ab_hbm, w2_hbm, out_hbm,
                idxa_all, idxb_all, idxa_x, idxb_x, a_v, b_v, w2_v, score_v,
                sema, semb):
    cid = lax.axis_index("c")
    sid = lax.axis_index("s")
    wid = cid * _NS + sid
    base = _ORPW * wid

    pltpu.sync_copy(idxa_hbm.at[pl.ds(base, _ORPW)], idxa_all)
    pltpu.sync_copy(idxb_hbm.at[pl.ds(base, _ORPW)], idxb_all)
    pltpu.sync_copy(w2_hbm, w2_v)
    w2q = [w2_v[pl.ds(q * 16, 16)] for q in range(_H // 16)]

    def _process(ia_row, ib_row, row):
        ca = pltpu.async_copy(ab_hbm.at[ia_row], a_v, sema)
        cb = pltpu.async_copy(ab_hbm.at[ib_row], b_v, semb)
        ca.wait()
        cb.wait()

        @pl.loop(0, 8)
        def _grp(t):
            lanes = t * 16 + lax.iota(jnp.int32, 16)
            acc = jnp.zeros((16,), jnp.float32)
            for k in range(_H):
                kk = jnp.full((16,), k, jnp.int32)
                ga = plsc.load_gather(a_v, [lanes, kk])
                gb = plsc.load_gather(b_v, [lanes, kk + _H])
                acc = acc + jnp.maximum(ga + gb, 0.0) * w2q[k // 16][k % 16]
            score_v[t] = acc

        pltpu.sync_copy(score_v, out_hbm.at[row])

    @pl.loop(0, _ORPW)
    def _main(r):
        _process(idxa_all.at[r, 0], idxb_all.at[r, 0], base + r)

    @pl.when(wid < _OXTRA)
    def _tail():
        row = _ORPW * _NW + wid
        pltpu.sync_copy(idxa_hbm.at[pl.ds(row, 1)], idxa_x)
        pltpu.sync_copy(idxb_hbm.at[pl.ds(row, 1)], idxb_x)
        _process(idxa_x.at[0, 0], idxb_x.at[0, 0], row)


@functools.lru_cache(maxsize=None)
def _od_pass_kernel():
    return pl.kernel(
        _od_sc_body,
        out_type=jax.ShapeDtypeStruct((_OROWS, 8, 16), jnp.float32),
        mesh=_sc_mesh(),
        compiler_params=pltpu.CompilerParams(needs_layout_passes=False),
        scratch_types=[
            pltpu.VMEM((_ORPW, 1, 128), jnp.int32),
            pltpu.VMEM((_ORPW, 1, 128), jnp.int32),
            pltpu.VMEM((1, 1, 128), jnp.int32),
            pltpu.VMEM((1, 1, 128), jnp.int32),
            pltpu.VMEM((128, 2 * _H), jnp.float32),
            pltpu.VMEM((128, 2 * _H), jnp.float32),
            pltpu.VMEM((_H,), jnp.float32),
            pltpu.VMEM((8, 16), jnp.float32),
            pltpu.SemaphoreType.DMA,
            pltpu.SemaphoreType.DMA,
        ],
    )


def kernel(node_features, edge_index, edge_features, od_pairs, traffic_stats,
           W_node, b_node, W_edge, b_edge,
           Wm0, bm0, Wu0, bu0, Wm1, bm1, Wu1, bu1,
           Ws1, bs1, Ws2, bs2,
           Wk1, bk1, Wk2, bk2, Wk3, bk3):
    ei = edge_index.astype(jnp.int32)
    src_rows = ei[0].reshape(_EROWS, 1, 128)
    dst_rows = ei[1].reshape(_EROWS, 1, 128)

    odp = jnp.pad(od_pairs.astype(jnp.int32), ((0, _NODP - _NOD), (0, 0)))
    idxa = odp[:, 0].reshape(_OROWS, 1, 128)
    idxb = odp[:, 1].reshape(_OROWS, 1, 128)

    bn = b_node.reshape(1, _H)
    be = b_edge.reshape(1, _H)

    h0, g0 = _node_embed(node_features, W_node, bn, Wm0[:_H])
    c_all = _edge_embed(edge_features, W_edge, be, Wm0[_H:], bm0.reshape(1, _H),
                        Wm1[_H:], bm1.reshape(1, _H))

    agg0 = _edge_pass_kernel(0)(src_rows, dst_rows, c_all, g0)
    h1, g1 = _update0(h0, agg0, Wu0[:_H], Wu0[_H:], bu0.reshape(1, _H), Wm1[:_H])

    agg1 = _edge_pass_kernel(4)(src_rows, dst_rows, c_all, g1)
    ab_nodes, kc, ki = _update1(
        h1, agg1, Wu1[:_H], Wu1[_H:], bu1.reshape(1, _H),
        Ws1[:_H], Ws1[_H:], bs1.reshape(1, _H), traffic_stats.reshape(1, 4),
        Wk1[:_H], Wk1[_H:], bk1.reshape(1, 32), Wk2, bk2.reshape(1, 16),
        Wk3, bk3.reshape(1, 1))

    odout = _od_pass_kernel()(idxa, idxb, ab_nodes, Ws2.reshape(_H))
    scores = odout.reshape(-1)[:_NOD] + bs2[0]
    return scores, kc[0, 0], ki[0, 0]


# speedup vs baseline: 3.2730x; 1.2593x over previous
import functools

import jax
import jax.numpy as jnp
from jax import lax
from jax.experimental import pallas as pl
from jax.experimental.pallas import tpu as pltpu
from jax.experimental.pallas import tpu_sc as plsc

_NN = 10000
_NE = 320000
_DF = 128
_DE = 16
_H = 64
_NOD = 50000

_NC = 2
_NS = 16
_NW = _NC * _NS

_EROWS = _NE // 128
_ERPW = _EROWS // _NW
_EXTRA = _EROWS - _ERPW * _NW
_TPS = 624
_TREM = _NN - _TPS * _NS

_NODP = 50176
_OROWS = _NODP // 128
_ORPW = _OROWS // _NW
_OXTRA = _OROWS - _ORPW * _NW


@functools.lru_cache(maxsize=None)
def _sc_mesh():
    return plsc.VectorSubcoreMesh(core_axis_name="c", subcore_axis_name="s",
                                  num_cores=_NC, num_subcores=_NS)


def _node_embed_body(x_ref, wn_ref, bn_ref, wma_ref, h_ref, g_ref):
    h = jnp.maximum(jnp.dot(x_ref[...], wn_ref[...],
                            preferred_element_type=jnp.float32) + bn_ref[...], 0.0)
    h_ref[...] = h
    g = jnp.dot(h, wma_ref[...], preferred_element_type=jnp.float32)
    g_ref[...] = jnp.concatenate([g, jnp.zeros_like(g)], axis=1)


def _node_embed(x, wn, bn, wma):
    return pl.pallas_call(
        _node_embed_body,
        out_shape=(jax.ShapeDtypeStruct((_NN, _H), jnp.float32),
                   jax.ShapeDtypeStruct((_NN, 2 * _H), jnp.float32)),
    )(x, wn, bn, wma)


_EBLK = 4000


def _edge_embed_body(ef_ref, we_ref, be_ref, wb0_ref, bm0_ref, wb1_ref, bm1_ref,
                     c0_ref, c1_ref):
    e = jnp.maximum(jnp.dot(ef_ref[...], we_ref[...],
                            preferred_element_type=jnp.float32) + be_ref[...], 0.0)
    c0_ref[...] = jnp.dot(e, wb0_ref[...], preferred_element_type=jnp.float32) + bm0_ref[...]
    c1_ref[...] = jnp.dot(e, wb1_ref[...], preferred_element_type=jnp.float32) + bm1_ref[...]


def _edge_embed(ef, we, be, wb0, bm0, wb1, bm1):
    nblk = _NE // _EBLK
    return pl.pallas_call(
        _edge_embed_body,
        grid=(nblk,),
        in_specs=[pl.BlockSpec((_EBLK, _DE), lambda i: (i, 0)),
                  pl.BlockSpec((_DE, _H), lambda i: (0, 0)),
                  pl.BlockSpec((1, _H), lambda i: (0, 0)),
                  pl.BlockSpec((_H, _H), lambda i: (0, 0)),
                  pl.BlockSpec((1, _H), lambda i: (0, 0)),
                  pl.BlockSpec((_H, _H), lambda i: (0, 0)),
                  pl.BlockSpec((1, _H), lambda i: (0, 0))],
        out_specs=(pl.BlockSpec((_EBLK, _H), lambda i: (i, 0)),
                   pl.BlockSpec((_EBLK, _H), lambda i: (i, 0))),
        out_shape=(jax.ShapeDtypeStruct((_NE, _H), jnp.float32),
                   jax.ShapeDtypeStruct((_NE, _H), jnp.float32)),
    )(ef, we, be, wb0, bm0, wb1, bm1)


def _update0_body(h_ref, agg_ref, wua_ref, wub_ref, bu_ref, wma_ref, h1_ref, g1_ref):
    aggsum = agg_ref[0] + agg_ref[1]
    h1 = jnp.maximum(jnp.dot(h_ref[...], wua_ref[...], preferred_element_type=jnp.float32)
                     + jnp.dot(aggsum, wub_ref[...], preferred_element_type=jnp.float32)
                     + bu_ref[...], 0.0)
    h1_ref[...] = h1
    g1 = jnp.dot(h1, wma_ref[...], preferred_element_type=jnp.float32)
    g1_ref[...] = jnp.concatenate([g1, jnp.zeros_like(g1)], axis=1)


def _update0(h, agg, wua, wub, bu, wma):
    return pl.pallas_call(
        _update0_body,
        out_shape=(jax.ShapeDtypeStruct((_NN, _H), jnp.float32),
                   jax.ShapeDtypeStruct((_NN, 2 * _H), jnp.float32)),
    )(h, agg, wua, wub, bu, wma)


def _update1_body(h_ref, agg_ref, wua_ref, wub_ref, bu_ref,
                  ws1a_ref, ws1b_ref, bs1_ref, ts_ref,
                  wk1a_ref, wk1b_ref, bk1_ref, wk2_ref, bk2_ref, wk3_ref, bk3_ref,
                  ab_ref, kc_ref, ki_ref):
    aggsum = agg_ref[0] + agg_ref[1]
    h2 = jnp.maximum(jnp.dot(h_ref[...], wua_ref[...], preferred_element_type=jnp.float32)
                     + jnp.dot(aggsum, wub_ref[...], preferred_element_type=jnp.float32)
                     + bu_ref[...], 0.0)
    a = jnp.dot(h2, ws1a_ref[...], preferred_element_type=jnp.float32) + bs1_ref[...]
    b = jnp.dot(h2, ws1b_ref[...], preferred_element_type=jnp.float32)
    ab_ref[...] = jnp.concatenate([a, b], axis=1)
    ge = jnp.sum(h2, axis=0, keepdims=True) * (1.0 / _NN)
    x1 = jnp.maximum(jnp.dot(ge, wk1a_ref[...], preferred_element_type=jnp.float32)
                     + jnp.dot(ts_ref[...], wk1b_ref[...], preferred_element_type=jnp.float32)
                     + bk1_ref[...], 0.0)
    x2 = jnp.maximum(jnp.dot(x1, wk2_ref[...], preferred_element_type=jnp.float32)
                     + bk2_ref[...], 0.0)
    raw = jnp.dot(x2, wk3_ref[...], preferred_element_type=jnp.float32) + bk3_ref[...]
    kc = 1.0 + jax.nn.sigmoid(raw) * 49.0
    kc_ref[...] = kc
    ki_ref[...] = jnp.clip(jnp.round(kc), 1.0, 50.0)


def _update1(h, agg, wua, wub, bu, ws1a, ws1b, bs1, ts,
             wk1a, wk1b, bk1, wk2, bk2, wk3, bk3):
    return pl.pallas_call(
        _update1_body,
        out_shape=(jax.ShapeDtypeStruct((_NN, 2 * _H), jnp.float32),
                   jax.ShapeDtypeStruct((1, 1), jnp.float32),
                   jax.ShapeDtypeStruct((1, 1), jnp.float32)),
    )(h, agg, wua, wub, bu, ws1a, ws1b, bs1, ts,
      wk1a, wk1b, bk1, wk2, bk2, wk3, bk3)


def _edge_sc_body(src_hbm, dst_hbm, c_hbm, g_hbm, out_hbm,
                  agg_s, src_x, dst_x, c_v, rows_v,
                  gsem0, gsem1, csem0, csem1, ssem0, ssem1):
    cid = lax.axis_index("c")
    sid = lax.axis_index("s")
    wid = cid * _NS + sid
    base = _ERPW * wid
    gsems = (gsem0, gsem1)
    csems = (csem0, csem1)
    ssems = (ssem0, ssem1)

    @pl.loop(0, 128)
    def _zero(i):
        for q in range(_H // 16):
            c_v[0, i, pl.ds(q * 16, 16)] = jnp.zeros((16,), jnp.float32)

    for k, sz in ((0, 128), (128, 128), (256, 128), (384, 128), (512, 112)):
        pltpu.sync_copy(c_v.at[0, pl.ds(0, sz)],
                        agg_s.at[pl.ds(sid * _TPS + k, sz)])

    @pl.when(sid == 0)
    def _zrem():
        pltpu.sync_copy(c_v.at[0, pl.ds(0, _TREM)],
                        agg_s.at[pl.ds(_TPS * _NS, _TREM)])

    plsc.subcore_barrier()

    def _launch(row, b):
        pltpu.sync_copy(src_hbm.at[pl.ds(row, 1)], src_x.at[pl.ds(b, 1)])
        pltpu.sync_copy(dst_hbm.at[pl.ds(row, 1)], dst_x.at[pl.ds(b, 1)])
        pltpu.async_copy(g_hbm.at[src_x.at[b, 0]], rows_v.at[b], gsems[b])
        pltpu.async_copy(c_hbm.at[pl.ds(row * 128, 128)], c_v.at[b], csems[b])

    def _wait_scatter(b):
        pltpu.make_async_copy(c_v.at[b], agg_s.at[dst_x.at[b, 0]],
                              ssems[b]).wait()

    def _finish(row, b):
        pltpu.make_async_copy(g_hbm.at[src_x.at[b, 0]], rows_v.at[b],
                              gsems[b]).wait()
        pltpu.make_async_copy(c_hbm.at[pl.ds(row * 128, 128)], c_v.at[b],
                              csems[b]).wait()

        @pl.loop(0, 128, unroll=2)
        def _relu(j):
            for q in range(_H // 16):
                s = pl.ds(q * 16, 16)
                c_v[b, j, s] = jnp.maximum(rows_v[b, j, s] + c_v[b, j, s], 0.0)

        pltpu.async_copy(c_v.at[b], agg_s.at[dst_x.at[b, 0]], ssems[b],
                         add=True)

    _launch(base + 0, 0)
    _launch(base + 1, 1)

    @pl.loop(0, _ERPW // 2)
    def _main(k):
        for b in (0, 1):
            r = 2 * k + b
            _finish(base + r, b)

            @pl.when(r + 2 < _ERPW)
            def _next():
                _wait_scatter(b)
                _launch(base + r + 2, b)

    for b in (0, 1):
        _wait_scatter(b)

    @pl.when(wid < _EXTRA)
    def _tail():
        row = _ERPW * _NW + wid
        _launch(row, 0)
        _finish(row, 0)
        _wait_scatter(0)

    plsc.subcore_barrier()
    pltpu.sync_copy(agg_s.at[pl.ds(sid * _TPS, _TPS)],
                    out_hbm.at[cid, pl.ds(sid * _TPS, _TPS)])

    @pl.when(sid == 0)
    def _orem():
        pltpu.sync_copy(agg_s.at[pl.ds(_TPS * _NS, _TREM)],
                        out_hbm.at[cid, pl.ds(_TPS * _NS, _TREM)])


@functools.lru_cache(maxsize=None)
def _edge_pass_kernel():
    return pl.kernel(
        _edge_sc_body,
        out_type=jax.ShapeDtypeStruct((_NC, _NN, _H), jnp.float32),
        mesh=_sc_mesh(),
        scratch_types=[
            pltpu.VMEM_SHARED((_NN, _H), jnp.float32),
            pltpu.VMEM((2, 1, 128), jnp.int32),
            pltpu.VMEM((2, 1, 128), jnp.int32),
            pltpu.VMEM((2, 128, _H), jnp.float32),
            pltpu.VMEM((2, 128, 2 * _H), jnp.float32),
            pltpu.SemaphoreType.DMA,
            pltpu.SemaphoreType.DMA,
            pltpu.SemaphoreType.DMA,
            pltpu.SemaphoreType.DMA,
            pltpu.SemaphoreType.DMA,
            pltpu.SemaphoreType.DMA,
        ],
    )


def _od_sc_body(idxa_hbm, idxb_hbm, ab_hbm, w2_hbm, out_hbm,
                idxa_x, idxb_x, a_v, b_v, w2_v, score_v,
                sa0, sa1, sb0, sb1):
    cid = lax.axis_index("c")
    sid = lax.axis_index("s")
    wid = cid * _NS + sid
    base = _ORPW * wid
    sas = (sa0, sa1)
    sbs = (sb0, sb1)

    pltpu.sync_copy(w2_hbm, w2_v)
    w2q = [w2_v[pl.ds(q * 16, 16)] for q in range(_H // 16)]

    def _launch(row, b):
        pltpu.sync_copy(idxa_hbm.at[pl.ds(row, 1)], idxa_x.at[pl.ds(b, 1)])
        pltpu.sync_copy(idxb_hbm.at[pl.ds(row, 1)], idxb_x.at[pl.ds(b, 1)])
        pltpu.async_copy(ab_hbm.at[idxa_x.at[b, 0]], a_v.at[b], sas[b])
        pltpu.async_copy(ab_hbm.at[idxb_x.at[b, 0]], b_v.at[b], sbs[b])

    def _finish(row, b):
        pltpu.make_async_copy(ab_hbm.at[idxa_x.at[b, 0]], a_v.at[b],
                              sas[b]).wait()
        pltpu.make_async_copy(ab_hbm.at[idxb_x.at[b, 0]], b_v.at[b],
                              sbs[b]).wait()

        @pl.loop(0, 8)
        def _grp(t):
            lanes = t * 16 + lax.iota(jnp.int32, 16)
            acc = jnp.zeros((16,), jnp.float32)
            for k in range(_H):
                kk = jnp.full((16,), k, jnp.int32)
                ga = plsc.load_gather(a_v.at[b], [lanes, kk])
                gb = plsc.load_gather(b_v.at[b], [lanes, kk + _H])
                acc = acc + jnp.maximum(ga + gb, 0.0) * w2q[k // 16][k % 16]
            score_v[b, t] = acc

        pltpu.sync_copy(score_v.at[b], out_hbm.at[row])

    _launch(base + 0, 0)
    _launch(base + 1, 1)

    @pl.loop(0, _ORPW // 2)
    def _main(k):
        for b in (0, 1):
            r = 2 * k + b
            _finish(base + r, b)

            @pl.when(r + 2 < _ORPW)
            def _next():
                _launch(base + r + 2, b)

    @pl.when(wid < _OXTRA)
    def _tail():
        row = _ORPW * _NW + wid
        _launch(row, 0)
        _finish(row, 0)


@functools.lru_cache(maxsize=None)
def _od_pass_kernel():
    return pl.kernel(
        _od_sc_body,
        out_type=jax.ShapeDtypeStruct((_OROWS, 8, 16), jnp.float32),
        mesh=_sc_mesh(),
        compiler_params=pltpu.CompilerParams(needs_layout_passes=False),
        scratch_types=[
            pltpu.VMEM((2, 1, 128), jnp.int32),
            pltpu.VMEM((2, 1, 128), jnp.int32),
            pltpu.VMEM((2, 128, 2 * _H), jnp.float32),
            pltpu.VMEM((2, 128, 2 * _H), jnp.float32),
            pltpu.VMEM((_H,), jnp.float32),
            pltpu.VMEM((2, 8, 16), jnp.float32),
            pltpu.SemaphoreType.DMA,
            pltpu.SemaphoreType.DMA,
            pltpu.SemaphoreType.DMA,
            pltpu.SemaphoreType.DMA,
        ],
    )


def kernel(node_features, edge_index, edge_features, od_pairs, traffic_stats,
           W_node, b_node, W_edge, b_edge,
           Wm0, bm0, Wu0, bu0, Wm1, bm1, Wu1, bu1,
           Ws1, bs1, Ws2, bs2,
           Wk1, bk1, Wk2, bk2, Wk3, bk3):
    ei = edge_index.astype(jnp.int32)
    src_rows = ei[0].reshape(_EROWS, 1, 128)
    dst_rows = ei[1].reshape(_EROWS, 1, 128)

    odp = jnp.pad(od_pairs.astype(jnp.int32), ((0, _NODP - _NOD), (0, 0)))
    idxa = odp[:, 0].reshape(_OROWS, 1, 128)
    idxb = odp[:, 1].reshape(_OROWS, 1, 128)

    bn = b_node.reshape(1, _H)
    be = b_edge.reshape(1, _H)

    h0, g0 = _node_embed(node_features, W_node, bn, Wm0[:_H])
    c0, c1 = _edge_embed(edge_features, W_edge, be, Wm0[_H:], bm0.reshape(1, _H),
                         Wm1[_H:], bm1.reshape(1, _H))

    agg0 = _edge_pass_kernel()(src_rows, dst_rows, c0, g0)
    h1, g1 = _update0(h0, agg0, Wu0[:_H], Wu0[_H:], bu0.reshape(1, _H), Wm1[:_H])

    agg1 = _edge_pass_kernel()(src_rows, dst_rows, c1, g1)
    ab_nodes, kc, ki = _update1(
        h1, agg1, Wu1[:_H], Wu1[_H:], bu1.reshape(1, _H),
        Ws1[:_H], Ws1[_H:], bs1.reshape(1, _H), traffic_stats.reshape(1, 4),
        Wk1[:_H], Wk1[_H:], bk1.reshape(1, 32), Wk2, bk2.reshape(1, 16),
        Wk3, bk3.reshape(1, 1))

    odout = _od_pass_kernel()(idxa, idxb, ab_nodes, Ws2.reshape(_H))
    scores = odout.reshape(-1)[:_NOD] + bs2[0]
    return scores, kc[0, 0], ki[0, 0]


# speedup vs baseline: 3.3136x; 1.0124x over previous
import functools

import jax
import jax.numpy as jnp
from jax import lax
from jax.experimental import pallas as pl
from jax.experimental.pallas import tpu as pltpu
from jax.experimental.pallas import tpu_sc as plsc

_NN = 10000
_NE = 320000
_DF = 128
_DE = 16
_H = 64
_NOD = 50000

_NC = 2
_NS = 16
_NW = _NC * _NS

_EROWS = _NE // 128
_ERPW = _EROWS // _NW
_EXTRA = _EROWS - _ERPW * _NW
_TPS = 624
_TREM = _NN - _TPS * _NS

_NODP = 50176
_OROWS = _NODP // 128
_ORPW = _OROWS // _NW
_OXTRA = _OROWS - _ORPW * _NW


@functools.lru_cache(maxsize=None)
def _sc_mesh():
    return plsc.VectorSubcoreMesh(core_axis_name="c", subcore_axis_name="s",
                                  num_cores=_NC, num_subcores=_NS)


def _node_embed_body(x_ref, wn_ref, bn_ref, wma_ref, h_ref, g_ref):
    h = jnp.maximum(jnp.dot(x_ref[...], wn_ref[...],
                            preferred_element_type=jnp.float32) + bn_ref[...], 0.0)
    h_ref[...] = h
    g = jnp.dot(h, wma_ref[...], preferred_element_type=jnp.float32)
    g_ref[...] = jnp.concatenate([g, jnp.zeros_like(g)], axis=1)


def _node_embed(x, wn, bn, wma):
    return pl.pallas_call(
        _node_embed_body,
        out_shape=(jax.ShapeDtypeStruct((_NN, _H), jnp.float32),
                   jax.ShapeDtypeStruct((_NN, 2 * _H), jnp.float32)),
    )(x, wn, bn, wma)


_EBLK = 4000


def _edge_embed_body(ef_ref, we_ref, be_ref, wb_ref, bm_ref, c_ref):
    e = jnp.maximum(jnp.dot(ef_ref[...], we_ref[...],
                            preferred_element_type=jnp.float32) + be_ref[...], 0.0)
    c_ref[...] = jnp.dot(e, wb_ref[...], preferred_element_type=jnp.float32) + bm_ref[...]


def _edge_embed(ef, we, be, wb, bm):
    nblk = _NE // _EBLK
    return pl.pallas_call(
        _edge_embed_body,
        grid=(nblk,),
        in_specs=[pl.BlockSpec((_EBLK, _DE), lambda i: (i, 0)),
                  pl.BlockSpec((_DE, _H), lambda i: (0, 0)),
                  pl.BlockSpec((1, _H), lambda i: (0, 0)),
                  pl.BlockSpec((_H, _H), lambda i: (0, 0)),
                  pl.BlockSpec((1, _H), lambda i: (0, 0))],
        out_specs=pl.BlockSpec((_EBLK, _H), lambda i: (i, 0)),
        out_shape=jax.ShapeDtypeStruct((_NE, _H), jnp.float32),
    )(ef, we, be, wb, bm)


def _update0_body(h_ref, agg_ref, wua_ref, wub_ref, bu_ref, wma_ref, h1_ref, g1_ref):
    aggsum = agg_ref[0] + agg_ref[1]
    h1 = jnp.maximum(jnp.dot(h_ref[...], wua_ref[...], preferred_element_type=jnp.float32)
                     + jnp.dot(aggsum, wub_ref[...], preferred_element_type=jnp.float32)
                     + bu_ref[...], 0.0)
    h1_ref[...] = h1
    g1 = jnp.dot(h1, wma_ref[...], preferred_element_type=jnp.float32)
    g1_ref[...] = jnp.concatenate([g1, jnp.zeros_like(g1)], axis=1)


def _update0(h, agg, wua, wub, bu, wma):
    return pl.pallas_call(
        _update0_body,
        out_shape=(jax.ShapeDtypeStruct((_NN, _H), jnp.float32),
                   jax.ShapeDtypeStruct((_NN, 2 * _H), jnp.float32)),
    )(h, agg, wua, wub, bu, wma)


def _update1_body(h_ref, agg_ref, wua_ref, wub_ref, bu_ref,
                  ws1a_ref, ws1b_ref, bs1_ref, ts_ref,
                  wk1a_ref, wk1b_ref, bk1_ref, wk2_ref, bk2_ref, wk3_ref, bk3_ref,
                  ab_ref, kc_ref, ki_ref):
    aggsum = agg_ref[0] + agg_ref[1]
    h2 = jnp.maximum(jnp.dot(h_ref[...], wua_ref[...], preferred_element_type=jnp.float32)
                     + jnp.dot(aggsum, wub_ref[...], preferred_element_type=jnp.float32)
                     + bu_ref[...], 0.0)
    a = jnp.dot(h2, ws1a_ref[...], preferred_element_type=jnp.float32) + bs1_ref[...]
    b = jnp.dot(h2, ws1b_ref[...], preferred_element_type=jnp.float32)
    ab_ref[...] = jnp.concatenate([a, b], axis=1)
    ge = jnp.sum(h2, axis=0, keepdims=True) * (1.0 / _NN)
    x1 = jnp.maximum(jnp.dot(ge, wk1a_ref[...], preferred_element_type=jnp.float32)
                     + jnp.dot(ts_ref[...], wk1b_ref[...], preferred_element_type=jnp.float32)
                     + bk1_ref[...], 0.0)
    x2 = jnp.maximum(jnp.dot(x1, wk2_ref[...], preferred_element_type=jnp.float32)
                     + bk2_ref[...], 0.0)
    raw = jnp.dot(x2, wk3_ref[...], preferred_element_type=jnp.float32) + bk3_ref[...]
    kc = 1.0 + jax.nn.sigmoid(raw) * 49.0
    kc_ref[...] = kc
    ki_ref[...] = jnp.clip(jnp.round(kc), 1.0, 50.0)


def _update1(h, agg, wua, wub, bu, ws1a, ws1b, bs1, ts,
             wk1a, wk1b, bk1, wk2, bk2, wk3, bk3):
    return pl.pallas_call(
        _update1_body,
        out_shape=(jax.ShapeDtypeStruct((_NN, 2 * _H), jnp.float32),
                   jax.ShapeDtypeStruct((1, 1), jnp.float32),
                   jax.ShapeDtypeStruct((1, 1), jnp.float32)),
    )(h, agg, wua, wub, bu, ws1a, ws1b, bs1, ts,
      wk1a, wk1b, bk1, wk2, bk2, wk3, bk3)


def _edge_sc_body(src_hbm, dst_hbm, c_hbm, g_hbm, out_hbm,
                  agg_s, src_x, dst_x, c_v, rows_v,
                  gsem0, gsem1, csem0, csem1, ssem0, ssem1):
    cid = lax.axis_index("c")
    sid = lax.axis_index("s")
    wid = cid * _NS + sid
    base = _ERPW * wid
    gsems = (gsem0, gsem1)
    csems = (csem0, csem1)
    ssems = (ssem0, ssem1)

    @pl.loop(0, 128)
    def _zero(i):
        for q in range(_H // 16):
            c_v[0, i, pl.ds(q * 16, 16)] = jnp.zeros((16,), jnp.float32)

    for k, sz in ((0, 128), (128, 128), (256, 128), (384, 128), (512, 112)):
        pltpu.sync_copy(c_v.at[0, pl.ds(0, sz)],
                        agg_s.at[pl.ds(sid * _TPS + k, sz)])

    @pl.when(sid == 0)
    def _zrem():
        pltpu.sync_copy(c_v.at[0, pl.ds(0, _TREM)],
                        agg_s.at[pl.ds(_TPS * _NS, _TREM)])

    plsc.subcore_barrier()

    def _launch(row, b):
        pltpu.sync_copy(src_hbm.at[pl.ds(row, 1)], src_x.at[pl.ds(b, 1)])
        pltpu.sync_copy(dst_hbm.at[pl.ds(row, 1)], dst_x.at[pl.ds(b, 1)])
        pltpu.async_copy(g_hbm.at[src_x.at[b, 0]], rows_v.at[b], gsems[b])
        pltpu.async_copy(c_hbm.at[pl.ds(row * 128, 128)], c_v.at[b], csems[b])

    def _wait_scatter(b):
        pltpu.make_async_copy(c_v.at[b], agg_s.at[dst_x.at[b, 0]],
                              ssems[b]).wait()

    def _finish(row, b):
        pltpu.make_async_copy(g_hbm.at[src_x.at[b, 0]], rows_v.at[b],
                              gsems[b]).wait()
        pltpu.make_async_copy(c_hbm.at[pl.ds(row * 128, 128)], c_v.at[b],
                              csems[b]).wait()

        @pl.loop(0, 128, unroll=2)
        def _relu(j):
            for q in range(_H // 16):
                s = pl.ds(q * 16, 16)
                c_v[b, j, s] = jnp.maximum(rows_v[b, j, s] + c_v[b, j, s], 0.0)

        pltpu.async_copy(c_v.at[b], agg_s.at[dst_x.at[b, 0]], ssems[b],
                         add=True)

    _launch(base + 0, 0)
    _launch(base + 1, 1)

    @pl.loop(0, _ERPW // 2)
    def _main(k):
        for b in (0, 1):
            r = 2 * k + b
            _finish(base + r, b)

            @pl.when(r + 2 < _ERPW)
            def _next():
                _wait_scatter(b)
                _launch(base + r + 2, b)

    for b in (0, 1):
        _wait_scatter(b)

    @pl.when(wid < _EXTRA)
    def _tail():
        row = _ERPW * _NW + wid
        _launch(row, 0)
        _finish(row, 0)
        _wait_scatter(0)

    plsc.subcore_barrier()
    pltpu.sync_copy(agg_s.at[pl.ds(sid * _TPS, _TPS)],
                    out_hbm.at[cid, pl.ds(sid * _TPS, _TPS)])

    @pl.when(sid == 0)
    def _orem():
        pltpu.sync_copy(agg_s.at[pl.ds(_TPS * _NS, _TREM)],
                        out_hbm.at[cid, pl.ds(_TPS * _NS, _TREM)])


@functools.lru_cache(maxsize=None)
def _edge_pass_kernel():
    return pl.kernel(
        _edge_sc_body,
        out_type=jax.ShapeDtypeStruct((_NC, _NN, _H), jnp.float32),
        mesh=_sc_mesh(),
        scratch_types=[
            pltpu.VMEM_SHARED((_NN, _H), jnp.float32),
            pltpu.VMEM((2, 1, 128), jnp.int32),
            pltpu.VMEM((2, 1, 128), jnp.int32),
            pltpu.VMEM((2, 128, _H), jnp.float32),
            pltpu.VMEM((2, 128, 2 * _H), jnp.float32),
            pltpu.SemaphoreType.DMA,
            pltpu.SemaphoreType.DMA,
            pltpu.SemaphoreType.DMA,
            pltpu.SemaphoreType.DMA,
            pltpu.SemaphoreType.DMA,
            pltpu.SemaphoreType.DMA,
        ],
    )


def _od_sc_body(idxa_hbm, idxb_hbm, ab_hbm, w2_hbm, out_hbm,
                idxa_all, idxb_all, a_v, b_v, w2_v, score_v,
                sa0, sa1, sb0, sb1, osem):
    cid = lax.axis_index("c")
    sid = lax.axis_index("s")
    wid = cid * _NS + sid
    base = _ORPW * wid
    sas = (sa0, sa1)
    sbs = (sb0, sb1)

    pltpu.sync_copy(w2_hbm, w2_v)
    for r in range(_ORPW):
        pltpu.async_copy(idxa_hbm.at[pl.ds(base + r, 1)],
                         idxa_all.at[pl.ds(r, 1)], osem)
        pltpu.async_copy(idxb_hbm.at[pl.ds(base + r, 1)],
                         idxb_all.at[pl.ds(r, 1)], osem)
    for r in range(_ORPW):
        pltpu.make_async_copy(idxa_hbm.at[pl.ds(base + r, 1)],
                              idxa_all.at[pl.ds(r, 1)], osem).wait()
        pltpu.make_async_copy(idxb_hbm.at[pl.ds(base + r, 1)],
                              idxb_all.at[pl.ds(r, 1)], osem).wait()
    w2q = [w2_v[pl.ds(q * 16, 16)] for q in range(_H // 16)]

    def _launch(r, b):
        pltpu.async_copy(ab_hbm.at[idxa_all.at[r, 0]], a_v.at[b], sas[b])
        pltpu.async_copy(ab_hbm.at[idxb_all.at[r, 0]], b_v.at[b], sbs[b])

    def _finish(r, b, orow):
        pltpu.make_async_copy(ab_hbm.at[idxa_all.at[r, 0]], a_v.at[b],
                              sas[b]).wait()
        pltpu.make_async_copy(ab_hbm.at[idxb_all.at[r, 0]], b_v.at[b],
                              sbs[b]).wait()

        @pl.loop(0, 8)
        def _grp(t):
            lanes = t * 16 + lax.iota(jnp.int32, 16)
            acc = jnp.zeros((16,), jnp.float32)
            for k in range(_H):
                kk = jnp.full((16,), k, jnp.int32)
                ga = plsc.load_gather(a_v.at[b], [lanes, kk])
                gb = plsc.load_gather(b_v.at[b], [lanes, kk + _H])
                acc = acc + jnp.maximum(ga + gb, 0.0) * w2q[k // 16][k % 16]
            score_v[r, t] = acc

        pltpu.async_copy(score_v.at[pl.ds(r, 1)], out_hbm.at[pl.ds(orow, 1)],
                         osem)

    _launch(0, 0)
    _launch(1, 1)

    @pl.loop(0, _ORPW // 2)
    def _main(k):
        for b in (0, 1):
            r = 2 * k + b
            _finish(r, b, base + r)

            @pl.when(r + 2 < _ORPW)
            def _next():
                _launch(r + 2, b)

    for r in range(_ORPW):
        pltpu.make_async_copy(score_v.at[pl.ds(r, 1)],
                              out_hbm.at[pl.ds(base + r, 1)], osem).wait()

    @pl.when(wid < _OXTRA)
    def _tail():
        row = _ORPW * _NW + wid
        pltpu.sync_copy(idxa_hbm.at[pl.ds(row, 1)], idxa_all.at[pl.ds(0, 1)])
        pltpu.sync_copy(idxb_hbm.at[pl.ds(row, 1)], idxb_all.at[pl.ds(0, 1)])
        _launch(0, 0)
        _finish(0, 0, row)
        pltpu.make_async_copy(score_v.at[pl.ds(0, 1)],
                              out_hbm.at[pl.ds(row, 1)], osem).wait()


@functools.lru_cache(maxsize=None)
def _od_pass_kernel():
    return pl.kernel(
        _od_sc_body,
        out_type=jax.ShapeDtypeStruct((_OROWS, 8, 16), jnp.float32),
        mesh=_sc_mesh(),
        compiler_params=pltpu.CompilerParams(needs_layout_passes=False),
        scratch_types=[
            pltpu.VMEM((_ORPW, 1, 128), jnp.int32),
            pltpu.VMEM((_ORPW, 1, 128), jnp.int32),
            pltpu.VMEM((2, 128, 2 * _H), jnp.float32),
            pltpu.VMEM((2, 128, 2 * _H), jnp.float32),
            pltpu.VMEM((_H,), jnp.float32),
            pltpu.VMEM((_ORPW, 8, 16), jnp.float32),
            pltpu.SemaphoreType.DMA,
            pltpu.SemaphoreType.DMA,
            pltpu.SemaphoreType.DMA,
            pltpu.SemaphoreType.DMA,
            pltpu.SemaphoreType.DMA,
        ],
    )


def kernel(node_features, edge_index, edge_features, od_pairs, traffic_stats,
           W_node, b_node, W_edge, b_edge,
           Wm0, bm0, Wu0, bu0, Wm1, bm1, Wu1, bu1,
           Ws1, bs1, Ws2, bs2,
           Wk1, bk1, Wk2, bk2, Wk3, bk3):
    ei = edge_index.astype(jnp.int32)
    src_rows = ei[0].reshape(_EROWS, 1, 128)
    dst_rows = ei[1].reshape(_EROWS, 1, 128)

    odp = jnp.pad(od_pairs.astype(jnp.int32), ((0, _NODP - _NOD), (0, 0)))
    idxa = odp[:, 0].reshape(_OROWS, 1, 128)
    idxb = odp[:, 1].reshape(_OROWS, 1, 128)

    bn = b_node.reshape(1, _H)
    be = b_edge.reshape(1, _H)

    h0, g0 = _node_embed(node_features, W_node, bn, Wm0[:_H])
    c0 = _edge_embed(edge_features, W_edge, be, Wm0[_H:], bm0.reshape(1, _H))

    agg0 = _edge_pass_kernel()(src_rows, dst_rows, c0, g0)
    c1 = _edge_embed(edge_features, W_edge, be, Wm1[_H:], bm1.reshape(1, _H))
    h1, g1 = _update0(h0, agg0, Wu0[:_H], Wu0[_H:], bu0.reshape(1, _H), Wm1[:_H])

    agg1 = _edge_pass_kernel()(src_rows, dst_rows, c1, g1)
    ab_nodes, kc, ki = _update1(
        h1, agg1, Wu1[:_H], Wu1[_H:], bu1.reshape(1, _H),
        Ws1[:_H], Ws1[_H:], bs1.reshape(1, _H), traffic_stats.reshape(1, 4),
        Wk1[:_H], Wk1[_H:], bk1.reshape(1, 32), Wk2, bk2.reshape(1, 16),
        Wk3, bk3.reshape(1, 1))

    odout = _od_pass_kernel()(idxa, idxb, ab_nodes, Ws2.reshape(_H))
    scores = odout.reshape(-1)[:_NOD] + bs2[0]
    return scores, kc[0, 0], ki[0, 0]
